# Initial kernel scaffold; baseline (speedup 1.0000x reference)
import jax
import jax.numpy as jnp
from jax.experimental import pallas as pl


def kernel(x, edge_index, edge_attr, pairs, W_emb, b_emb, Wl0, bl0, Wr0, Wl1, bl1, Wr1, W1, b1, W2, b2):
    raise NotImplementedError("write your pallas kernel here")


# speedup vs baseline: 4.9777x; 4.9777x over previous
import functools

import jax
import jax.numpy as jnp
from jax import lax
from jax.experimental import pallas as pl
from jax.experimental.pallas import tpu as pltpu
from jax.experimental.pallas import tpu_sc as plsc

N = 10000
E = 320000
D = 128
H = 128
P = 8192

NC = 2
NS = 16
NW = NC * NS
CH = 128
NCH = 79
EW = NCH * CH
EP = NW * EW
NPAD = 10240
RPT = NPAD // NS

PW = (2 * P) // NW
PCH = PW // CH

_MESH = dict(core_axis_name="c", subcore_axis_name="s")


def _make_agg(compute_deg):
    out_type = [jax.ShapeDtypeStruct((NC, NPAD, D), jnp.float32)]
    if compute_deg:
        out_type.append(jax.ShapeDtypeStruct((NC, NPAD), jnp.float32))

    scratch = [
        pltpu.VMEM((NCH, CH), jnp.int32),
        pltpu.VMEM((NCH, CH), jnp.int32),
        pltpu.VMEM((CH, D), jnp.float32),
        pltpu.VMEM((CH,), jnp.float32),
        pltpu.VMEM_SHARED((NPAD, D), jnp.float32),
        pltpu.VMEM_SHARED((NPAD,), jnp.float32),
        pltpu.SemaphoreType.DMA,
    ]

    @functools.partial(
        pl.kernel,
        mesh=plsc.VectorSubcoreMesh(**_MESH),
        out_type=out_type,
        scratch_types=scratch,
    )
    def agg_kernel(table, srcp, dstp, zeros2, zeros1, ones_h, *rest):
        if compute_deg:
            agg_out, deg_out = rest[0], rest[1]
            rest = rest[2:]
        else:
            agg_out = rest[0]
            rest = rest[1:]
        src_v, dst_v, rows_v, ones_v, acc_sh, deg_sh, sem = rest

        c = lax.axis_index("c")
        s = lax.axis_index("s")
        wid = c * NS + s

        pltpu.sync_copy(zeros2, acc_sh.at[pl.ds(s * RPT, RPT)])
        if compute_deg:
            pltpu.sync_copy(zeros1, deg_sh.at[pl.ds(s * RPT, RPT)])
            pltpu.sync_copy(ones_h, ones_v)
        pltpu.sync_copy(srcp.at[wid], src_v)
        pltpu.sync_copy(dstp.at[wid], dst_v)
        plsc.subcore_barrier()

        def body(j, carry):
            pltpu.async_copy(table.at[src_v.at[j]], rows_v, sem).wait()
            pltpu.sync_copy(rows_v, acc_sh.at[dst_v.at[j]], add=True)
            if compute_deg:
                pltpu.sync_copy(ones_v, deg_sh.at[dst_v.at[j]], add=True)
            return carry

        lax.fori_loop(0, NCH, body, 0)
        plsc.subcore_barrier()

        pltpu.sync_copy(acc_sh.at[pl.ds(s * RPT, RPT)],
                        agg_out.at[c, pl.ds(s * RPT, RPT)])
        if compute_deg:
            pltpu.sync_copy(deg_sh.at[pl.ds(s * RPT, RPT)],
                            deg_out.at[c, pl.ds(s * RPT, RPT)])

    return agg_kernel


_agg_with_deg = _make_agg(True)
_agg_no_deg = _make_agg(False)


@functools.partial(
    pl.kernel,
    mesh=plsc.VectorSubcoreMesh(**_MESH),
    out_type=jax.ShapeDtypeStruct((2 * P, D), jnp.float32),
    scratch_types=[
        pltpu.VMEM((PCH, CH), jnp.int32),
        pltpu.VMEM((CH, D), jnp.float32),
        pltpu.SemaphoreType.DMA,
    ],
)
def _pair_gather(table, idxp, out, idx_v, rows_v, sem):
    c = lax.axis_index("c")
    s = lax.axis_index("s")
    wid = c * NS + s
    pltpu.sync_copy(idxp.at[wid], idx_v)

    def body(j, carry):
        pltpu.async_copy(table.at[idx_v.at[j]], rows_v, sem).wait()
        pltpu.sync_copy(rows_v, out.at[pl.ds(wid * PW + j * CH, CH)])
        return carry

    lax.fori_loop(0, PCH, body, 0)


def _emb_body(x_ref, w_ref, b_ref, o_ref):
    o_ref[...] = lax.dot_general(
        x_ref[...], w_ref[...], (((1,), (1,)), ((), ())),
        preferred_element_type=jnp.float32) + b_ref[...]


def _conv_body(act, aggp_ref, degp_ref, h_ref, wl_ref, bl_ref, wr_ref, o_ref):
    agg = aggp_ref[0] + aggp_ref[1]
    deg = jnp.maximum(degp_ref[0] + degp_ref[1], 1.0)
    agg = agg / deg
    o = lax.dot_general(agg, wl_ref[...], (((1,), (1,)), ((), ())),
                        preferred_element_type=jnp.float32) + bl_ref[...]
    o = o + lax.dot_general(h_ref[...], wr_ref[...], (((1,), (1,)), ((), ())),
                            preferred_element_type=jnp.float32)
    if act:
        o = jnp.where(o > 0, o, 0.1 * o)
    o_ref[...] = o


def _head_body(hp_ref, w1_ref, b1_ref, w2_ref, b2_ref, o_ref):
    u = lax.dot_general(hp_ref[...], w1_ref[...], (((1,), (1,)), ((), ())),
                        preferred_element_type=jnp.float32) + b1_ref[...]
    u = jnp.where(u > 0, u, 0.1 * u)
    o_ref[...] = lax.dot_general(
        u, w2_ref[...], (((1,), (1,)), ((), ())),
        preferred_element_type=jnp.float32) + b2_ref[...]


def _emb(x, w, b):
    return pl.pallas_call(
        _emb_body,
        out_shape=jax.ShapeDtypeStruct((N, D), jnp.float32),
    )(x, w, b.reshape(1, H))


def _conv(act, aggp, degp, h, wl, bl, wr):
    return pl.pallas_call(
        functools.partial(_conv_body, act),
        out_shape=jax.ShapeDtypeStruct((N, H), jnp.float32),
    )(aggp, degp, h, wl, bl.reshape(1, H), wr)


def _head(hp, w1, b1, w2, b2):
    w2r = jnp.broadcast_to(w2.reshape(1, H), (H, H))
    b2r = jnp.broadcast_to(b2.reshape(1, 1), (1, H))
    o = pl.pallas_call(
        _head_body,
        out_shape=jax.ShapeDtypeStruct((P, H), jnp.float32),
    )(hp, w1, b1.reshape(1, H), w2r, b2r)
    return o[:, :1]


def kernel(x, edge_index, edge_attr, pairs, W_emb, b_emb, Wl0, bl0, Wr0,
           Wl1, bl1, Wr1, W1, b1, W2, b2):
    src = edge_index[0]
    dst = edge_index[1]
    srcp = jnp.pad(src, (0, EP - E)).reshape(NW, NCH, CH)
    dstp = jnp.pad(dst, (0, EP - E), constant_values=N).reshape(NW, NCH, CH)
    zeros2 = jnp.zeros((RPT, D), jnp.float32)
    zeros1 = jnp.zeros((RPT,), jnp.float32)
    ones_h = jnp.ones((CH,), jnp.float32)
    idxp = pairs.reshape(NW, PCH, CH)

    h0 = _emb(x, W_emb, b_emb)
    aggp0, degp = _agg_with_deg(h0, srcp, dstp, zeros2, zeros1, ones_h)
    deg = degp[:, :N, None]
    h1 = _conv(True, aggp0[:, :N], deg, h0, Wl0, bl0, Wr0)
    (aggp1,) = _agg_no_deg(h1, srcp, dstp, zeros2, zeros1, ones_h)
    h2 = _conv(False, aggp1[:, :N], deg, h1, Wl1, bl1, Wr1)
    rows = _pair_gather(h2, idxp)
    hp = rows.reshape(P, 2 * H)
    return _head(hp, W1, b1, W2, b2)
